# baseline (speedup 1.0000x reference)
import functools

import jax
import jax.numpy as jnp
from jax import lax
from jax.experimental import pallas as pl
from jax.experimental.pallas import tpu as pltpu
from jax.experimental.pallas import tpu_sc as plsc

N_QUERY = 800_000
N_MEM = 1_600_000

NC, NS, L = 2, 16, 16
NW = NC * NS

BLK = 1024
ROWS = BLK // 128
GRP = BLK // L

M_PER_T = N_MEM // NS
MFULL = M_PER_T // BLK
MTAIL = M_PER_T - MFULL * BLK

Q_PER_W = N_QUERY // NW
QFULL = Q_PER_W // BLK
QTAIL = Q_PER_W - QFULL * BLK

TBL = 589_824
TSLICE = TBL // NS
ZBLK = 4096
ZITER = TSLICE // ZBLK

SENT = 550_000


def _i32(x):
    return jnp.int32(x)


_GDN = lax.GatherDimensionNumbers(
    offset_dims=(), collapsed_slice_dims=(0,), start_index_map=(0,))


def _permute(v, idx):
    return lax.gather(v, idx[:, None], _GDN, slice_sizes=(1,),
                      mode=lax.GatherScatterMode.PROMISE_IN_BOUNDS)


def _evens2():
    lane = lax.iota(jnp.int32, L)
    return (lane & _i32(7)) << _i32(1)


def _halfmask():
    return lax.iota(jnp.int32, L) < _i32(8)


def _lo16(buf_ref, word_off):
    v0 = buf_ref[pl.ds(word_off, 16)]
    v1 = buf_ref[pl.ds(word_off + 16, 16)]
    idx = _evens2()
    g0 = _permute(v0, idx)
    g1 = _permute(v1, idx)
    return jnp.where(_halfmask(), g0, g1)


def _keys_full(src_ref, dst_ref, kidx_ref, ngroups):
    for g in range(ngroups):
        sv = _lo16(src_ref, g * 32)
        dv = _lo16(dst_ref, g * 32)
        kidx_ref[g // 8, pl.ds((g % 8) * 16, 16)] = sv * _i32(10) + dv


def _keys_fill(kidx_ref, g_lo, g_hi, value):
    filler = jnp.full((L,), value, jnp.int32)
    for g in range(g_lo, g_hi):
        kidx_ref[g // 8, pl.ds((g % 8) * 16, 16)] = filler


def _sc_kernel(qs, qd, ms, md, pos16, out,
               table, sbuf, dbuf, kidx, vals, qval, zbuf, pbuf, sem):
    c = lax.axis_index("c")
    s = lax.axis_index("s")
    wid = s * _i32(NC) + c

    def zinit(i, _):
        zbuf[pl.ds(i * _i32(16), 16)] = jnp.zeros((16,), jnp.float32)
        return 0
    lax.fori_loop(_i32(0), _i32(ZBLK // 16), zinit, 0)
    for r in range(ZITER):
        pltpu.sync_copy(zbuf, table.at[pl.ds(s * _i32(TSLICE) + _i32(r * ZBLK), ZBLK)])

    pltpu.sync_copy(pos16, pbuf)
    pv = pbuf[...]
    for j in range(ROWS):
        for i in range(8):
            vals[j, pl.ds(i * 16, 16)] = pv

    plsc.subcore_barrier()

    def scat_block(b, _):
        base = pl.multiple_of((s * _i32(M_PER_T) + b * _i32(BLK)) * _i32(2), 8)
        pltpu.sync_copy(ms.at[pl.ds(base, 2 * BLK)], sbuf)
        pltpu.sync_copy(md.at[pl.ds(base, 2 * BLK)], dbuf)
        _keys_full(sbuf, dbuf, kidx, GRP)
        copies = [pltpu.async_copy(vals.at[_i32(j)], table.at[kidx.at[_i32(j)]], sem)
                  for j in range(ROWS)]
        for cp in copies:
            cp.wait()
        return 0
    lax.fori_loop(_i32(0), _i32(MFULL), scat_block, 0)

    tbase = pl.multiple_of((s * _i32(M_PER_T) + _i32(MFULL * BLK)) * _i32(2), 8)
    pltpu.sync_copy(ms.at[pl.ds(tbase, 2 * MTAIL)], sbuf.at[pl.ds(0, 2 * MTAIL)])
    pltpu.sync_copy(md.at[pl.ds(tbase, 2 * MTAIL)], dbuf.at[pl.ds(0, 2 * MTAIL)])
    _keys_full(sbuf, dbuf, kidx, MTAIL // L)
    _keys_fill(kidx, MTAIL // L, GRP, SENT)
    copies = [pltpu.async_copy(vals.at[_i32(j)], table.at[kidx.at[_i32(j)]], sem)
              for j in range(ROWS)]
    for cp in copies:
        cp.wait()

    plsc.subcore_barrier()

    def gath_block(b, _):
        ebase = pl.multiple_of(wid * _i32(Q_PER_W) + b * _i32(BLK), 8)
        wbase = pl.multiple_of(ebase * _i32(2), 8)
        pltpu.sync_copy(qs.at[pl.ds(wbase, 2 * BLK)], sbuf)
        pltpu.sync_copy(qd.at[pl.ds(wbase, 2 * BLK)], dbuf)
        _keys_full(sbuf, dbuf, kidx, GRP)
        copies = [pltpu.async_copy(table.at[kidx.at[_i32(j)]],
                                   qval.at[pl.ds(j * 128, 128)], sem)
                  for j in range(ROWS)]
        for cp in copies:
            cp.wait()
        pltpu.sync_copy(qval, out.at[pl.ds(ebase, BLK)])
        return 0
    lax.fori_loop(_i32(0), _i32(QFULL), gath_block, 0)

    qebase = pl.multiple_of(wid * _i32(Q_PER_W) + _i32(QFULL * BLK), 8)
    qwbase = pl.multiple_of(qebase * _i32(2), 8)
    pltpu.sync_copy(qs.at[pl.ds(qwbase, 2 * QTAIL)], sbuf.at[pl.ds(0, 2 * QTAIL)])
    pltpu.sync_copy(qd.at[pl.ds(qwbase, 2 * QTAIL)], dbuf.at[pl.ds(0, 2 * QTAIL)])
    ng = QTAIL // L
    rem = QTAIL - ng * L
    _keys_full(sbuf, dbuf, kidx, ng)
    v0s = sbuf[pl.ds(ng * 32, 16)]
    v0d = dbuf[pl.ds(ng * 32, 16)]
    idx = _evens2()
    sv = _permute(v0s, idx)
    dv = _permute(v0d, idx)
    lane = lax.iota(jnp.int32, L)
    kidx[ng // 8, pl.ds((ng % 8) * 16, 16)] = jnp.where(
        lane < _i32(rem), sv * _i32(10) + dv, _i32(0))
    _keys_fill(kidx, ng + 1, GRP, 0)
    copies = [pltpu.async_copy(table.at[kidx.at[_i32(j)]],
                               qval.at[pl.ds(j * 128, 128)], sem)
              for j in range(ROWS)]
    for cp in copies:
        cp.wait()
    pltpu.sync_copy(qval.at[pl.ds(0, QTAIL)], out.at[pl.ds(qebase, QTAIL)])


@functools.partial(
    pl.kernel,
    mesh=plsc.VectorSubcoreMesh(core_axis_name="c", subcore_axis_name="s",
                                num_cores=NC),
    out_type=jax.ShapeDtypeStruct((N_QUERY,), jnp.float32),
    scratch_types=[
        pltpu.VMEM_SHARED((TBL,), jnp.float32),
        pltpu.VMEM((2 * BLK,), jnp.int32),
        pltpu.VMEM((2 * BLK,), jnp.int32),
        pltpu.VMEM((ROWS, 128), jnp.int32),
        pltpu.VMEM((ROWS, 128), jnp.float32),
        pltpu.VMEM((BLK,), jnp.float32),
        pltpu.VMEM((ZBLK,), jnp.float32),
        pltpu.VMEM((16,), jnp.float32),
        pltpu.SemaphoreType.DMA,
    ],
)
def _edgebank_sc(qs, qd, ms, md, pos16, out,
                 table, sbuf, dbuf, kidx, vals, qval, zbuf, pbuf, sem):
    _sc_kernel(qs, qd, ms, md, pos16, out,
               table, sbuf, dbuf, kidx, vals, qval, zbuf, pbuf, sem)


def kernel(query_edge_indices, mem_edge_index, pos_prob):
    q32 = lax.bitcast_convert_type(query_edge_indices, jnp.int32)
    m32 = lax.bitcast_convert_type(mem_edge_index, jnp.int32)
    qs = q32[0].reshape(-1)
    qd = q32[1].reshape(-1)
    ms = m32[0].reshape(-1)
    md = m32[1].reshape(-1)
    pos16 = jnp.broadcast_to(pos_prob.astype(jnp.float32), (16,))
    return _edgebank_sc(qs, qd, ms, md, pos16)


# speedup vs baseline: 8.5935x; 8.5935x over previous
import functools

import jax
import jax.numpy as jnp
from jax import lax
from jax.experimental import pallas as pl
from jax.experimental.pallas import tpu as pltpu
from jax.experimental.pallas import tpu_sc as plsc

N_QUERY = 800_000
N_MEM = 1_600_000

NC, NS, L = 2, 16, 16
NW = NC * NS

BLK = 1024
ROWS = BLK // 128
GRP = BLK // L

MBLKS = N_MEM // BLK
MEXTRA = MBLKS % NS
MTAIL = N_MEM - MBLKS * BLK

QBLKS = N_QUERY // BLK
QEXTRA = QBLKS % NW
QTAIL = N_QUERY - QBLKS * BLK

TBL = 589_824
TSLICE = TBL // NS
ZBLK = 4096
ZITER = TSLICE // ZBLK

SENT = 550_000


def _i32(x):
    return jnp.int32(x)


def _keys_full(src_ref, dst_ref, kidx_ref, ngroups):
    for g in range(ngroups):
        sv = src_ref[pl.ds(g * 16, 16)]
        dv = dst_ref[pl.ds(g * 16, 16)]
        kidx_ref[g // 8, pl.ds((g % 8) * 16, 16)] = sv * _i32(10) + dv


def _keys_fill(kidx_ref, g_lo, g_hi, value):
    filler = jnp.full((L,), value, jnp.int32)
    for g in range(g_lo, g_hi):
        kidx_ref[g // 8, pl.ds((g % 8) * 16, 16)] = filler


def _sc_kernel(q2, m2, pos16, out,
               table, sbuf, dbuf, kidx, vals, qval, zbuf, pbuf, sem):
    c = lax.axis_index("c")
    s = lax.axis_index("s")
    wid = s * _i32(NC) + c
    qs, qd = q2.at[_i32(0)], q2.at[_i32(1)]
    ms, md = m2.at[_i32(0)], m2.at[_i32(1)]

    def zinit(i, _):
        zbuf[pl.ds(i * _i32(16), 16)] = jnp.zeros((16,), jnp.float32)
        return 0
    lax.fori_loop(_i32(0), _i32(ZBLK // 16), zinit, 0)
    for r in range(ZITER):
        pltpu.sync_copy(zbuf, table.at[pl.ds(s * _i32(TSLICE) + _i32(r * ZBLK), ZBLK)])

    pltpu.sync_copy(pos16, pbuf)
    pv = pbuf[...]
    for j in range(ROWS):
        for i in range(8):
            vals[j, pl.ds(i * 16, 16)] = pv

    plsc.subcore_barrier()

    def scat_block(b, _):
        base = pl.multiple_of((b * _i32(NS) + s) * _i32(BLK), BLK)
        pltpu.sync_copy(ms.at[pl.ds(base, BLK)], sbuf)
        pltpu.sync_copy(md.at[pl.ds(base, BLK)], dbuf)
        _keys_full(sbuf, dbuf, kidx, GRP)
        copies = [pltpu.async_copy(vals.at[_i32(j)], table.at[kidx.at[_i32(j)]], sem)
                  for j in range(ROWS)]
        for cp in copies:
            cp.wait()
        return 0
    mblk = _i32(MBLKS // NS) + jnp.where(s < _i32(MEXTRA), _i32(1), _i32(0))
    lax.fori_loop(_i32(0), mblk, scat_block, 0)

    @pl.when(s == _i32(NS - 1))
    def _mem_tail():
        tbase = _i32(MBLKS * BLK)
        pltpu.sync_copy(ms.at[pl.ds(tbase, MTAIL)], sbuf.at[pl.ds(0, MTAIL)])
        pltpu.sync_copy(md.at[pl.ds(tbase, MTAIL)], dbuf.at[pl.ds(0, MTAIL)])
        _keys_full(sbuf, dbuf, kidx, MTAIL // L)
        _keys_fill(kidx, MTAIL // L, GRP, SENT)
        copies = [pltpu.async_copy(vals.at[_i32(j)], table.at[kidx.at[_i32(j)]], sem)
                  for j in range(ROWS)]
        for cp in copies:
            cp.wait()

    plsc.subcore_barrier()

    def gath_block(b, _):
        base = pl.multiple_of((b * _i32(NW) + wid) * _i32(BLK), BLK)
        pltpu.sync_copy(qs.at[pl.ds(base, BLK)], sbuf)
        pltpu.sync_copy(qd.at[pl.ds(base, BLK)], dbuf)
        _keys_full(sbuf, dbuf, kidx, GRP)
        copies = [pltpu.async_copy(table.at[kidx.at[_i32(j)]],
                                   qval.at[pl.ds(j * 128, 128)], sem)
                  for j in range(ROWS)]
        for cp in copies:
            cp.wait()
        pltpu.sync_copy(qval, out.at[pl.ds(base, BLK)])
        return 0
    qblk = _i32(QBLKS // NW) + jnp.where(wid < _i32(QEXTRA), _i32(1), _i32(0))
    lax.fori_loop(_i32(0), qblk, gath_block, 0)

    @pl.when(wid == _i32(NW - 1))
    def _query_tail():
        qbase = _i32(QBLKS * BLK)
        pltpu.sync_copy(qs.at[pl.ds(qbase, QTAIL)], sbuf.at[pl.ds(0, QTAIL)])
        pltpu.sync_copy(qd.at[pl.ds(qbase, QTAIL)], dbuf.at[pl.ds(0, QTAIL)])
        _keys_full(sbuf, dbuf, kidx, QTAIL // L)
        _keys_fill(kidx, QTAIL // L, GRP, 0)
        copies = [pltpu.async_copy(table.at[kidx.at[_i32(j)]],
                                   qval.at[pl.ds(j * 128, 128)], sem)
                  for j in range(ROWS)]
        for cp in copies:
            cp.wait()
        pltpu.sync_copy(qval.at[pl.ds(0, QTAIL)], out.at[pl.ds(qbase, QTAIL)])


@functools.partial(
    pl.kernel,
    mesh=plsc.VectorSubcoreMesh(core_axis_name="c", subcore_axis_name="s",
                                num_cores=NC),
    out_type=jax.ShapeDtypeStruct((N_QUERY,), jnp.float32),
    scratch_types=[
        pltpu.VMEM_SHARED((TBL,), jnp.float32),
        pltpu.VMEM((BLK,), jnp.int32),
        pltpu.VMEM((BLK,), jnp.int32),
        pltpu.VMEM((ROWS, 128), jnp.int32),
        pltpu.VMEM((ROWS, 128), jnp.float32),
        pltpu.VMEM((BLK,), jnp.float32),
        pltpu.VMEM((ZBLK,), jnp.float32),
        pltpu.VMEM((16,), jnp.float32),
        pltpu.SemaphoreType.DMA,
    ],
)
def _edgebank_sc(q2, m2, pos16, out,
                 table, sbuf, dbuf, kidx, vals, qval, zbuf, pbuf, sem):
    _sc_kernel(q2, m2, pos16, out,
               table, sbuf, dbuf, kidx, vals, qval, zbuf, pbuf, sem)


def kernel(query_edge_indices, mem_edge_index, pos_prob):
    q = query_edge_indices.astype(jnp.int32)
    m = mem_edge_index.astype(jnp.int32)
    pos16 = jnp.broadcast_to(pos_prob.astype(jnp.float32), (16,))
    return _edgebank_sc(q, m, pos16)


# speedup vs baseline: 12.3316x; 1.4350x over previous
import functools

import jax
import jax.numpy as jnp
from jax import lax
from jax.experimental import pallas as pl
from jax.experimental.pallas import tpu as pltpu
from jax.experimental.pallas import tpu_sc as plsc

N_QUERY = 800_000
N_MEM = 1_600_000

NC, NS, L = 2, 16, 16
NW = NC * NS

BLK = 1024
ROWS = BLK // 128
GRP = BLK // L

MBLKS = N_MEM // BLK
MEXTRA = MBLKS % NS
MTAIL = N_MEM - MBLKS * BLK

QBLKS = N_QUERY // BLK
QEXTRA = QBLKS % NW
QTAIL = N_QUERY - QBLKS * BLK

TBL = 589_824
TSLICE = TBL // NS
ZBLK = 4096
ZITER = TSLICE // ZBLK

SENT = 550_000


def _i32(x):
    return jnp.int32(x)


def _keys_full(src_ref, dst_ref, kidx_ref, ngroups):
    for g in range(ngroups):
        sv = src_ref[pl.ds(g * 16, 16)]
        dv = dst_ref[pl.ds(g * 16, 16)]
        kidx_ref[g // 8, pl.ds((g % 8) * 16, 16)] = sv * _i32(10) + dv


def _keys_fill(kidx_ref, g_lo, g_hi, value):
    filler = jnp.full((L,), value, jnp.int32)
    for g in range(g_lo, g_hi):
        kidx_ref[g // 8, pl.ds((g % 8) * 16, 16)] = filler


def _sc_kernel(q2, m2, pos16, out,
               table, sbuf, dbuf, sbuf2, dbuf2, kidx, vals, qval, zbuf, pbuf,
               sem, lsem_a, lsem_b):
    c = lax.axis_index("c")
    s = lax.axis_index("s")
    wid = s * _i32(NC) + c
    qs, qd = q2.at[_i32(0)], q2.at[_i32(1)]
    ms, md = m2.at[_i32(0)], m2.at[_i32(1)]

    def zinit(i, _):
        zbuf[pl.ds(i * _i32(16), 16)] = jnp.zeros((16,), jnp.float32)
        return 0
    lax.fori_loop(_i32(0), _i32(ZBLK // 16), zinit, 0)
    for r in range(ZITER):
        pltpu.sync_copy(zbuf, table.at[pl.ds(s * _i32(TSLICE) + _i32(r * ZBLK), ZBLK)])

    pltpu.sync_copy(pos16, pbuf)
    pv = pbuf[...]
    for j in range(ROWS):
        for i in range(8):
            vals[j, pl.ds(i * 16, 16)] = pv

    plsc.subcore_barrier()

    mblk = _i32(MBLKS // NS) + jnp.where(s < _i32(MEXTRA), _i32(1), _i32(0))

    base0 = pl.multiple_of(s * _i32(BLK), BLK)
    pltpu.async_copy(ms.at[pl.ds(base0, BLK)], sbuf, lsem_a)
    pltpu.async_copy(md.at[pl.ds(base0, BLK)], dbuf, lsem_a)

    def _scat_half(b, cur_s, cur_d, cur_sem, nxt_s, nxt_d, nxt_sem):
        pltpu.make_async_copy(ms.at[pl.ds(0, BLK)], cur_s, cur_sem).wait()
        pltpu.make_async_copy(md.at[pl.ds(0, BLK)], cur_d, cur_sem).wait()

        @pl.when(b + _i32(1) < mblk)
        def _prefetch():
            nbase = pl.multiple_of(((b + _i32(1)) * _i32(NS) + s) * _i32(BLK), BLK)
            pltpu.async_copy(ms.at[pl.ds(nbase, BLK)], nxt_s, nxt_sem)
            pltpu.async_copy(md.at[pl.ds(nbase, BLK)], nxt_d, nxt_sem)

        _keys_full(cur_s, cur_d, kidx, GRP)
        copies = [pltpu.async_copy(vals.at[_i32(j)], table.at[kidx.at[_i32(j)]], sem)
                  for j in range(ROWS)]
        for cp in copies:
            cp.wait()

    def scat_block(b, _):
        @pl.when((b & _i32(1)) == _i32(0))
        def _even():
            _scat_half(b, sbuf, dbuf, lsem_a, sbuf2, dbuf2, lsem_b)

        @pl.when((b & _i32(1)) == _i32(1))
        def _odd():
            _scat_half(b, sbuf2, dbuf2, lsem_b, sbuf, dbuf, lsem_a)
        return 0
    lax.fori_loop(_i32(0), mblk, scat_block, 0)

    @pl.when(s == _i32(NS - 1))
    def _mem_tail():
        tbase = _i32(MBLKS * BLK)
        pltpu.sync_copy(ms.at[pl.ds(tbase, MTAIL)], sbuf.at[pl.ds(0, MTAIL)])
        pltpu.sync_copy(md.at[pl.ds(tbase, MTAIL)], dbuf.at[pl.ds(0, MTAIL)])
        _keys_full(sbuf, dbuf, kidx, MTAIL // L)
        _keys_fill(kidx, MTAIL // L, GRP, SENT)
        copies = [pltpu.async_copy(vals.at[_i32(j)], table.at[kidx.at[_i32(j)]], sem)
                  for j in range(ROWS)]
        for cp in copies:
            cp.wait()

    plsc.subcore_barrier()

    qblk = _i32(QBLKS // NW) + jnp.where(wid < _i32(QEXTRA), _i32(1), _i32(0))

    qbase0 = pl.multiple_of(wid * _i32(BLK), BLK)
    pltpu.async_copy(qs.at[pl.ds(qbase0, BLK)], sbuf, lsem_a)
    pltpu.async_copy(qd.at[pl.ds(qbase0, BLK)], dbuf, lsem_a)

    def _gath_half(b, cur_s, cur_d, cur_sem, nxt_s, nxt_d, nxt_sem):
        base = pl.multiple_of((b * _i32(NW) + wid) * _i32(BLK), BLK)
        pltpu.make_async_copy(qs.at[pl.ds(0, BLK)], cur_s, cur_sem).wait()
        pltpu.make_async_copy(qd.at[pl.ds(0, BLK)], cur_d, cur_sem).wait()

        @pl.when(b + _i32(1) < qblk)
        def _prefetch():
            nbase = pl.multiple_of(((b + _i32(1)) * _i32(NW) + wid) * _i32(BLK), BLK)
            pltpu.async_copy(qs.at[pl.ds(nbase, BLK)], nxt_s, nxt_sem)
            pltpu.async_copy(qd.at[pl.ds(nbase, BLK)], nxt_d, nxt_sem)

        _keys_full(cur_s, cur_d, kidx, GRP)
        copies = [pltpu.async_copy(table.at[kidx.at[_i32(j)]],
                                   qval.at[pl.ds(j * 128, 128)], sem)
                  for j in range(ROWS)]
        for cp in copies:
            cp.wait()
        pltpu.sync_copy(qval, out.at[pl.ds(base, BLK)])

    def gath_block(b, _):
        @pl.when((b & _i32(1)) == _i32(0))
        def _even():
            _gath_half(b, sbuf, dbuf, lsem_a, sbuf2, dbuf2, lsem_b)

        @pl.when((b & _i32(1)) == _i32(1))
        def _odd():
            _gath_half(b, sbuf2, dbuf2, lsem_b, sbuf, dbuf, lsem_a)
        return 0
    lax.fori_loop(_i32(0), qblk, gath_block, 0)

    @pl.when(wid == _i32(NW - 1))
    def _query_tail():
        qbase = _i32(QBLKS * BLK)
        pltpu.sync_copy(qs.at[pl.ds(qbase, QTAIL)], sbuf.at[pl.ds(0, QTAIL)])
        pltpu.sync_copy(qd.at[pl.ds(qbase, QTAIL)], dbuf.at[pl.ds(0, QTAIL)])
        _keys_full(sbuf, dbuf, kidx, QTAIL // L)
        _keys_fill(kidx, QTAIL // L, GRP, 0)
        copies = [pltpu.async_copy(table.at[kidx.at[_i32(j)]],
                                   qval.at[pl.ds(j * 128, 128)], sem)
                  for j in range(ROWS)]
        for cp in copies:
            cp.wait()
        pltpu.sync_copy(qval.at[pl.ds(0, QTAIL)], out.at[pl.ds(qbase, QTAIL)])


@functools.partial(
    pl.kernel,
    mesh=plsc.VectorSubcoreMesh(core_axis_name="c", subcore_axis_name="s",
                                num_cores=NC),
    out_type=jax.ShapeDtypeStruct((N_QUERY,), jnp.float32),
    scratch_types=[
        pltpu.VMEM_SHARED((TBL,), jnp.float32),
        pltpu.VMEM((BLK,), jnp.int32),
        pltpu.VMEM((BLK,), jnp.int32),
        pltpu.VMEM((BLK,), jnp.int32),
        pltpu.VMEM((BLK,), jnp.int32),
        pltpu.VMEM((ROWS, 128), jnp.int32),
        pltpu.VMEM((ROWS, 128), jnp.float32),
        pltpu.VMEM((BLK,), jnp.float32),
        pltpu.VMEM((ZBLK,), jnp.float32),
        pltpu.VMEM((16,), jnp.float32),
        pltpu.SemaphoreType.DMA,
        pltpu.SemaphoreType.DMA,
        pltpu.SemaphoreType.DMA,
    ],
)
def _edgebank_sc(q2, m2, pos16, out,
                 table, sbuf, dbuf, sbuf2, dbuf2, kidx, vals, qval, zbuf, pbuf,
                 sem, lsem_a, lsem_b):
    _sc_kernel(q2, m2, pos16, out,
               table, sbuf, dbuf, sbuf2, dbuf2, kidx, vals, qval, zbuf, pbuf,
               sem, lsem_a, lsem_b)


def kernel(query_edge_indices, mem_edge_index, pos_prob):
    q = query_edge_indices.astype(jnp.int32)
    m = mem_edge_index.astype(jnp.int32)
    pos16 = jnp.broadcast_to(pos_prob.astype(jnp.float32), (16,))
    return _edgebank_sc(q, m, pos16)


# speedup vs baseline: 12.9240x; 1.0480x over previous
import functools

import jax
import jax.numpy as jnp
from jax import lax
from jax.experimental import pallas as pl
from jax.experimental.pallas import tpu as pltpu
from jax.experimental.pallas import tpu_sc as plsc

N_QUERY = 800_000
N_MEM = 1_600_000

NC, NS, L = 2, 16, 16
NW = NC * NS

BLK = 2048
ROWS = BLK // 128
GRP = BLK // L

MBLKS = N_MEM // BLK
MEXTRA = MBLKS % NS
MTAIL = N_MEM - MBLKS * BLK

QBLKS = N_QUERY // BLK
QEXTRA = QBLKS % NW
QTAIL = N_QUERY - QBLKS * BLK

TBL = 589_824
TSLICE = TBL // NS
ZBLK = 4096
ZITER = TSLICE // ZBLK

SENT = 550_000


def _i32(x):
    return jnp.int32(x)


def _keys_full(src_ref, dst_ref, kidx_ref, ngroups):
    for g in range(ngroups):
        sv = src_ref[pl.ds(g * 16, 16)]
        dv = dst_ref[pl.ds(g * 16, 16)]
        kidx_ref[g // 8, pl.ds((g % 8) * 16, 16)] = sv * _i32(10) + dv


def _keys_fill(kidx_ref, g_lo, g_hi, value):
    filler = jnp.full((L,), value, jnp.int32)
    for g in range(g_lo, g_hi):
        kidx_ref[g // 8, pl.ds((g % 8) * 16, 16)] = filler


def _sc_kernel(q2, m2, pos16, out,
               table, sbuf, dbuf, sbuf2, dbuf2, kidx, vals, qval, zbuf, pbuf,
               sem, lsem_a, lsem_b):
    c = lax.axis_index("c")
    s = lax.axis_index("s")
    wid = s * _i32(NC) + c
    qs, qd = q2.at[_i32(0)], q2.at[_i32(1)]
    ms, md = m2.at[_i32(0)], m2.at[_i32(1)]

    def zinit(i, _):
        zbuf[pl.ds(i * _i32(16), 16)] = jnp.zeros((16,), jnp.float32)
        return 0
    lax.fori_loop(_i32(0), _i32(ZBLK // 16), zinit, 0)
    for r in range(ZITER):
        pltpu.sync_copy(zbuf, table.at[pl.ds(s * _i32(TSLICE) + _i32(r * ZBLK), ZBLK)])

    pltpu.sync_copy(pos16, pbuf)
    pv = pbuf[...]
    for j in range(ROWS):
        for i in range(8):
            vals[j, pl.ds(i * 16, 16)] = pv

    plsc.subcore_barrier()

    mblk = _i32(MBLKS // NS) + jnp.where(s < _i32(MEXTRA), _i32(1), _i32(0))

    base0 = pl.multiple_of(s * _i32(BLK), BLK)
    pltpu.async_copy(ms.at[pl.ds(base0, BLK)], sbuf, lsem_a)
    pltpu.async_copy(md.at[pl.ds(base0, BLK)], dbuf, lsem_a)

    def _scat_half(b, cur_s, cur_d, cur_sem, nxt_s, nxt_d, nxt_sem):
        pltpu.make_async_copy(ms.at[pl.ds(0, BLK)], cur_s, cur_sem).wait()
        pltpu.make_async_copy(md.at[pl.ds(0, BLK)], cur_d, cur_sem).wait()

        @pl.when(b + _i32(1) < mblk)
        def _prefetch():
            nbase = pl.multiple_of(((b + _i32(1)) * _i32(NS) + s) * _i32(BLK), BLK)
            pltpu.async_copy(ms.at[pl.ds(nbase, BLK)], nxt_s, nxt_sem)
            pltpu.async_copy(md.at[pl.ds(nbase, BLK)], nxt_d, nxt_sem)

        _keys_full(cur_s, cur_d, kidx, GRP)
        copies = [pltpu.async_copy(vals.at[_i32(j)], table.at[kidx.at[_i32(j)]], sem)
                  for j in range(ROWS)]
        for cp in copies:
            cp.wait()

    def scat_block(b, _):
        @pl.when((b & _i32(1)) == _i32(0))
        def _even():
            _scat_half(b, sbuf, dbuf, lsem_a, sbuf2, dbuf2, lsem_b)

        @pl.when((b & _i32(1)) == _i32(1))
        def _odd():
            _scat_half(b, sbuf2, dbuf2, lsem_b, sbuf, dbuf, lsem_a)
        return 0
    lax.fori_loop(_i32(0), mblk, scat_block, 0)

    @pl.when(s == _i32(NS - 1))
    def _mem_tail():
        tbase = _i32(MBLKS * BLK)
        pltpu.sync_copy(ms.at[pl.ds(tbase, MTAIL)], sbuf.at[pl.ds(0, MTAIL)])
        pltpu.sync_copy(md.at[pl.ds(tbase, MTAIL)], dbuf.at[pl.ds(0, MTAIL)])
        _keys_full(sbuf, dbuf, kidx, MTAIL // L)
        _keys_fill(kidx, MTAIL // L, GRP, SENT)
        copies = [pltpu.async_copy(vals.at[_i32(j)], table.at[kidx.at[_i32(j)]], sem)
                  for j in range(ROWS)]
        for cp in copies:
            cp.wait()

    plsc.subcore_barrier()

    qblk = _i32(QBLKS // NW) + jnp.where(wid < _i32(QEXTRA), _i32(1), _i32(0))

    qbase0 = pl.multiple_of(wid * _i32(BLK), BLK)
    pltpu.async_copy(qs.at[pl.ds(qbase0, BLK)], sbuf, lsem_a)
    pltpu.async_copy(qd.at[pl.ds(qbase0, BLK)], dbuf, lsem_a)

    def _gath_half(b, cur_s, cur_d, cur_sem, nxt_s, nxt_d, nxt_sem):
        base = pl.multiple_of((b * _i32(NW) + wid) * _i32(BLK), BLK)
        pltpu.make_async_copy(qs.at[pl.ds(0, BLK)], cur_s, cur_sem).wait()
        pltpu.make_async_copy(qd.at[pl.ds(0, BLK)], cur_d, cur_sem).wait()

        @pl.when(b + _i32(1) < qblk)
        def _prefetch():
            nbase = pl.multiple_of(((b + _i32(1)) * _i32(NW) + wid) * _i32(BLK), BLK)
            pltpu.async_copy(qs.at[pl.ds(nbase, BLK)], nxt_s, nxt_sem)
            pltpu.async_copy(qd.at[pl.ds(nbase, BLK)], nxt_d, nxt_sem)

        _keys_full(cur_s, cur_d, kidx, GRP)
        copies = [pltpu.async_copy(table.at[kidx.at[_i32(j)]],
                                   qval.at[pl.ds(j * 128, 128)], sem)
                  for j in range(ROWS)]
        for cp in copies:
            cp.wait()
        pltpu.sync_copy(qval, out.at[pl.ds(base, BLK)])

    def gath_block(b, _):
        @pl.when((b & _i32(1)) == _i32(0))
        def _even():
            _gath_half(b, sbuf, dbuf, lsem_a, sbuf2, dbuf2, lsem_b)

        @pl.when((b & _i32(1)) == _i32(1))
        def _odd():
            _gath_half(b, sbuf2, dbuf2, lsem_b, sbuf, dbuf, lsem_a)
        return 0
    lax.fori_loop(_i32(0), qblk, gath_block, 0)

    @pl.when(wid == _i32(NW - 1))
    def _query_tail():
        qbase = _i32(QBLKS * BLK)
        pltpu.sync_copy(qs.at[pl.ds(qbase, QTAIL)], sbuf.at[pl.ds(0, QTAIL)])
        pltpu.sync_copy(qd.at[pl.ds(qbase, QTAIL)], dbuf.at[pl.ds(0, QTAIL)])
        _keys_full(sbuf, dbuf, kidx, QTAIL // L)
        _keys_fill(kidx, QTAIL // L, GRP, 0)
        copies = [pltpu.async_copy(table.at[kidx.at[_i32(j)]],
                                   qval.at[pl.ds(j * 128, 128)], sem)
                  for j in range(ROWS)]
        for cp in copies:
            cp.wait()
        pltpu.sync_copy(qval.at[pl.ds(0, QTAIL)], out.at[pl.ds(qbase, QTAIL)])


@functools.partial(
    pl.kernel,
    mesh=plsc.VectorSubcoreMesh(core_axis_name="c", subcore_axis_name="s",
                                num_cores=NC),
    out_type=jax.ShapeDtypeStruct((N_QUERY,), jnp.float32),
    scratch_types=[
        pltpu.VMEM_SHARED((TBL,), jnp.float32),
        pltpu.VMEM((BLK,), jnp.int32),
        pltpu.VMEM((BLK,), jnp.int32),
        pltpu.VMEM((BLK,), jnp.int32),
        pltpu.VMEM((BLK,), jnp.int32),
        pltpu.VMEM((ROWS, 128), jnp.int32),
        pltpu.VMEM((ROWS, 128), jnp.float32),
        pltpu.VMEM((BLK,), jnp.float32),
        pltpu.VMEM((ZBLK,), jnp.float32),
        pltpu.VMEM((16,), jnp.float32),
        pltpu.SemaphoreType.DMA,
        pltpu.SemaphoreType.DMA,
        pltpu.SemaphoreType.DMA,
    ],
)
def _edgebank_sc(q2, m2, pos16, out,
                 table, sbuf, dbuf, sbuf2, dbuf2, kidx, vals, qval, zbuf, pbuf,
                 sem, lsem_a, lsem_b):
    _sc_kernel(q2, m2, pos16, out,
               table, sbuf, dbuf, sbuf2, dbuf2, kidx, vals, qval, zbuf, pbuf,
               sem, lsem_a, lsem_b)


def kernel(query_edge_indices, mem_edge_index, pos_prob):
    q = query_edge_indices.astype(jnp.int32)
    m = mem_edge_index.astype(jnp.int32)
    pos16 = jnp.broadcast_to(pos_prob.astype(jnp.float32), (16,))
    return _edgebank_sc(q, m, pos16)
